# Initial kernel scaffold; baseline (speedup 1.0000x reference)
import jax
import jax.numpy as jnp
from jax.experimental import pallas as pl


def kernel(obj_idx, rel_idx, rel_weights, segment_ids, len_non_zero, s_idx, ent_embeds, rel_embeds, W, b):
    raise NotImplementedError("write your pallas kernel here")


# speedup vs baseline: 1.4997x; 1.4997x over previous
import functools

import jax
import jax.numpy as jnp
import numpy as np
from jax import lax
from jax.experimental import pallas as pl
from jax.experimental.pallas import tpu as pltpu
from jax.experimental.pallas import tpu_sc as plsc

T = 32768
N_GROUPS = 5120
B = 512
SEQ_LEN = 10
H = 128
HC = H // 16

NC, NS = 2, 16
NW = NC * NS
GPW = N_GROUPS // NW
K = 128
SROWS = B // NW
SEG_PAD = 1 << 20

ACC_N = (GPW + 1) * 2 * H
CNT_N = (GPW + 1) * 16


def _sc_body(obj_hbm, rel_hbm, w2_hbm, seg_hbm, starts_hbm, ends_hbm,
             sidx_hbm, zeros_hbm, ent_hbm, rele_hbm,
             sums_out, cnt_out, sent_out,
             obj_v, rel_v, seg_v, w2_v, ent_rows, rel_rows,
             acc_v, cnt_v, b_v, sidx_v, srow):
    cid = lax.axis_index("c")
    sid = lax.axis_index("s")
    wid = cid * NS + sid
    g0 = wid * GPW

    pltpu.sync_copy(zeros_hbm.at[pl.ds(0, ACC_N)], acc_v)
    pltpu.sync_copy(zeros_hbm.at[pl.ds(0, CNT_N)], cnt_v)

    srow0 = pl.multiple_of(wid * SROWS, 8)
    pltpu.sync_copy(sidx_hbm.at[pl.ds(srow0, SROWS)], sidx_v)
    pltpu.sync_copy(ent_hbm.at[sidx_v], srow)
    pltpu.sync_copy(srow, sent_out.at[pl.ds(srow0, SROWS)])

    lanes = lax.iota(jnp.int32, 16)
    bchunk = cid
    pltpu.sync_copy(starts_hbm.at[pl.ds(bchunk * 16, 16)], b_v)
    sv = b_v[...]
    astart = jnp.max(jnp.where(lanes == sid, sv, 0))
    pltpu.sync_copy(ends_hbm.at[pl.ds(bchunk * 16, 16)], b_v)
    ev = b_v[...]
    end = jnp.max(jnp.where(lanes == sid, ev, 0))
    nblk = (end - astart + K - 1) // K

    one0 = jnp.where(lanes == 0, 1.0, 0.0)
    iota16 = lanes

    def blk_body(bi, carry):
        base = pl.multiple_of(astart + bi * K, 8)
        pltpu.sync_copy(obj_hbm.at[pl.ds(base, K)], obj_v)
        pltpu.sync_copy(rel_hbm.at[pl.ds(base, K)], rel_v)
        pltpu.sync_copy(seg_hbm.at[pl.ds(base, K)], seg_v)
        pltpu.sync_copy(w2_hbm.at[pl.ds(pl.multiple_of(base * 16, 8), K * 16)],
                        w2_v)
        pltpu.sync_copy(rele_hbm.at[rel_v], rel_rows)
        pltpu.sync_copy(ent_hbm.at[obj_v], ent_rows)

        def fact(i, c2):
            sc = seg_v[pl.ds((i // 16) * 16, 16)]
            segb = jnp.take(sc, jnp.full((16,), i % 16, jnp.int32))
            ls = segb - g0
            valid = (ls >= 0) & (ls < GPW)
            lsc = jnp.where(valid, ls, GPW)
            validf = jnp.where(valid, 1.0, 0.0)
            wv = w2_v[pl.ds(i * 16, 16)] * validf
            av = lsc * (2 * H) + iota16
            for c in range(HC):
                vals = rel_rows[i, pl.ds(c * 16, 16)] * wv
                plsc.addupdate_scatter(acc_v, [av + c * 16], vals)
            for c in range(HC):
                vals = ent_rows[i, pl.ds(c * 16, 16)] * wv
                plsc.addupdate_scatter(acc_v, [av + (H + c * 16)], vals)
            plsc.addupdate_scatter(cnt_v, [lsc * 16 + iota16], one0 * validf)
            return c2

        lax.fori_loop(0, K, fact, 0)
        return carry

    lax.fori_loop(0, nblk, blk_body, 0)

    pltpu.sync_copy(acc_v.at[pl.ds(0, GPW * 2 * H)],
                    sums_out.at[pl.ds(pl.multiple_of(g0 * 2 * H, 8),
                                      GPW * 2 * H)])
    pltpu.sync_copy(cnt_v.at[pl.ds(0, GPW * 16)],
                    cnt_out.at[pl.ds(pl.multiple_of(g0 * 16, 8), GPW * 16)])


_sc_agg = pl.kernel(
    _sc_body,
    out_type=[
        jax.ShapeDtypeStruct((N_GROUPS * 2 * H,), jnp.float32),
        jax.ShapeDtypeStruct((N_GROUPS * 16,), jnp.float32),
        jax.ShapeDtypeStruct((B, H), jnp.float32),
    ],
    mesh=plsc.VectorSubcoreMesh(core_axis_name="c", subcore_axis_name="s"),
    compiler_params=pltpu.CompilerParams(needs_layout_passes=False),
    scratch_types=[
        pltpu.VMEM((K,), jnp.int32),
        pltpu.VMEM((K,), jnp.int32),
        pltpu.VMEM((K,), jnp.int32),
        pltpu.VMEM((K * 16,), jnp.float32),
        pltpu.VMEM((K, H), jnp.float32),
        pltpu.VMEM((K, H), jnp.float32),
        pltpu.VMEM((ACC_N,), jnp.float32),
        pltpu.VMEM((CNT_N,), jnp.float32),
        pltpu.VMEM((16,), jnp.int32),
        pltpu.VMEM((SROWS,), jnp.int32),
        pltpu.VMEM((SROWS, H), jnp.float32),
    ],
)

GBLK = 640


def _tc_body(sums_ref, cnt_ref, sent_ref, lnr_ref, e_ref, wt_ref, b_ref,
             out_ref):
    cnt = jnp.maximum(cnt_ref[...], 1.0)
    mean = sums_ref[...] / cnt
    h = jnp.dot(mean, wt_ref[...], preferred_element_type=jnp.float32)
    h = jnp.maximum(h + b_ref[...], 0.0)
    s = jnp.dot(e_ref[...], sent_ref[...], preferred_element_type=jnp.float32)
    tmod = lax.rem(lax.broadcasted_iota(jnp.int32, (GBLK, 1), 0),
                   jnp.int32(SEQ_LEN))
    mask = (tmod.astype(jnp.float32) < lnr_ref[...]).astype(jnp.float32)
    out_ref[...] = jnp.concatenate([h, s], axis=1) * mask


_E = np.repeat(np.eye(GBLK // SEQ_LEN, dtype=np.float32), SEQ_LEN, axis=0)

_tc_pack = pl.pallas_call(
    _tc_body,
    grid=(N_GROUPS // GBLK,),
    in_specs=[
        pl.BlockSpec((GBLK, 2 * H), lambda i: (i, 0)),
        pl.BlockSpec((GBLK, 1), lambda i: (i, 0)),
        pl.BlockSpec((GBLK // SEQ_LEN, H), lambda i: (i, 0)),
        pl.BlockSpec((GBLK, 1), lambda i: (i, 0)),
        pl.BlockSpec((GBLK, GBLK // SEQ_LEN), lambda i: (0, 0)),
        pl.BlockSpec((2 * H, 2 * H), lambda i: (0, 0)),
        pl.BlockSpec((1, 2 * H), lambda i: (0, 0)),
    ],
    out_specs=pl.BlockSpec((GBLK, 3 * H), lambda i: (i, 0)),
    out_shape=jax.ShapeDtypeStruct((N_GROUPS, 3 * H), jnp.float32),
)


def kernel(obj_idx, rel_idx, rel_weights, segment_ids, len_non_zero, s_idx,
           ent_embeds, rel_embeds, W, b):
    obj_idx = obj_idx.astype(jnp.int32)
    rel_idx = rel_idx.astype(jnp.int32)
    seg = segment_ids.astype(jnp.int32)
    s_idx = s_idx.astype(jnp.int32)
    w = rel_weights.astype(jnp.float32)

    zpad_i = jnp.zeros((K,), jnp.int32)
    obj_p = jnp.concatenate([obj_idx, zpad_i])
    rel_p = jnp.concatenate([rel_idx, zpad_i])
    seg_p = jnp.concatenate([seg, jnp.full((K,), SEG_PAD, jnp.int32)])
    w2_p = jnp.concatenate(
        [jnp.broadcast_to(w[:, None], (T, 16)),
         jnp.zeros((K, 16), jnp.float32)]).reshape(-1)

    bounds = jnp.searchsorted(seg, jnp.arange(NW + 1, dtype=jnp.int32) * GPW)
    starts = ((bounds[:NW] // 8) * 8).astype(jnp.int32)
    ends = bounds[1:].astype(jnp.int32)

    sums, cnt, sent = _sc_agg(obj_p, rel_p, w2_p, seg_p, starts, ends,
                              s_idx, jnp.zeros((ACC_N,), jnp.float32),
                              ent_embeds, rel_embeds)
    sums = sums.reshape(N_GROUPS, 2 * H)
    cnt1 = cnt.reshape(N_GROUPS, 16)[:, :1]
    lnr = jnp.repeat(jnp.maximum(len_non_zero, 1).astype(jnp.float32),
                     SEQ_LEN)[:, None]
    out = _tc_pack(sums, cnt1, sent, lnr, jnp.asarray(_E), W.T, b[None, :])
    return out.reshape(B, SEQ_LEN, 3 * H)


# speedup vs baseline: 1.8822x; 1.2551x over previous
import functools

import jax
import jax.numpy as jnp
import numpy as np
from jax import lax
from jax.experimental import pallas as pl
from jax.experimental.pallas import tpu as pltpu
from jax.experimental.pallas import tpu_sc as plsc

T = 32768
N_GROUPS = 5120
B = 512
SEQ_LEN = 10
H = 128
HC = H // 16

NC, NS = 2, 16
NW = NC * NS
GPW = N_GROUPS // NW
K = 128
KC = K // 16
SROWS = B // NW
SEG_PAD = 1 << 20

ACC_N = (GPW + 1) * 2 * H
CNT_N = (GPW + 1) * 16


def _sc_body(pidx_hbm, starts_hbm, ends_hbm, sidx_hbm, zeros_hbm,
             ent_hbm, rele_hbm,
             sums_out, cnt_out, sent_out,
             pidx0, pidx1, rel0, rel1, ent0, ent1,
             acc_v, cnt_v, b_v, sidx_v, srow,
             sem_r0, sem_r1, sem_e0, sem_e1):
    cid = lax.axis_index("c")
    sid = lax.axis_index("s")
    wid = cid * NS + sid
    g0 = wid * GPW

    pltpu.sync_copy(zeros_hbm.at[pl.ds(0, ACC_N)], acc_v)
    pltpu.sync_copy(zeros_hbm.at[pl.ds(0, CNT_N)], cnt_v)

    srow0 = pl.multiple_of(wid * SROWS, 8)
    pltpu.sync_copy(sidx_hbm.at[pl.ds(srow0, SROWS)], sidx_v)
    pltpu.sync_copy(ent_hbm.at[sidx_v], srow)
    pltpu.sync_copy(srow, sent_out.at[pl.ds(srow0, SROWS)])

    lanes = lax.iota(jnp.int32, 16)
    pltpu.sync_copy(starts_hbm.at[pl.ds(cid * 16, 16)], b_v)
    astart = jnp.max(jnp.where(lanes == sid, b_v[...], 0))
    pltpu.sync_copy(ends_hbm.at[pl.ds(cid * 16, 16)], b_v)
    end = jnp.max(jnp.where(lanes == sid, b_v[...], 0))
    nblk = (end - astart + K - 1) // K

    iota16 = lanes

    def issue(bj, pidx_v, relb, entb, sem_r, sem_e):
        base = pl.multiple_of(astart + bj * K, K)
        pltpu.sync_copy(pidx_hbm.at[:, pl.ds(base, K)], pidx_v)
        pltpu.async_copy(rele_hbm.at[pidx_v.at[1]], relb, sem_r)
        pltpu.async_copy(ent_hbm.at[pidx_v.at[0]], entb, sem_e)

    def compute(pidx_v, relb, entb, sem_r, sem_e):
        pltpu.make_async_copy(rele_hbm.at[pidx_v.at[1]], relb, sem_r).wait()
        pltpu.make_async_copy(ent_hbm.at[pidx_v.at[0]], entb, sem_e).wait()

        def chunk(ci, c2):
            sc = pidx_v[2, pl.ds(ci * 16, 16)]
            wc = plsc.bitcast(pidx_v[3, pl.ds(ci * 16, 16)], jnp.float32)
            ls = sc - g0
            valid = (ls >= 0) & (ls < GPW)
            lsc = jnp.where(valid, ls, GPW)
            validf = jnp.where(valid, 1.0, 0.0)
            wz = wc * validf
            av = lsc * (2 * H)
            plsc.addupdate_scatter(cnt_v, [lsc * 16 + iota16], validf)
            for j in range(16):
                jf = jnp.full((16,), j, jnp.int32)
                wj = jnp.take(wz, jf)
                a = jnp.take(av, jf) + iota16
                r = ci * 16 + j
                for c in range(HC):
                    vals = relb[r, pl.ds(c * 16, 16)] * wj
                    plsc.addupdate_scatter(acc_v, [a], vals)
                    a = a + 16
                for c in range(HC):
                    vals = entb[r, pl.ds(c * 16, 16)] * wj
                    plsc.addupdate_scatter(acc_v, [a], vals)
                    a = a + 16
            return c2

        lax.fori_loop(0, KC, chunk, 0)

    @pl.when(nblk > 0)
    def _():
        issue(0, pidx0, rel0, ent0, sem_r0, sem_e0)

    def blk(bi, carry):
        nxt = bi + 1

        @pl.when((nxt < nblk) & (nxt % 2 == 0))
        def _():
            issue(nxt, pidx0, rel0, ent0, sem_r0, sem_e0)

        @pl.when((nxt < nblk) & (nxt % 2 == 1))
        def _():
            issue(nxt, pidx1, rel1, ent1, sem_r1, sem_e1)

        @pl.when(bi % 2 == 0)
        def _():
            compute(pidx0, rel0, ent0, sem_r0, sem_e0)

        @pl.when(bi % 2 == 1)
        def _():
            compute(pidx1, rel1, ent1, sem_r1, sem_e1)

        return carry

    lax.fori_loop(0, nblk, blk, 0)

    pltpu.sync_copy(acc_v.at[pl.ds(0, GPW * 2 * H)],
                    sums_out.at[pl.ds(pl.multiple_of(g0 * 2 * H, 8),
                                      GPW * 2 * H)])
    pltpu.sync_copy(cnt_v.at[pl.ds(0, GPW * 16)],
                    cnt_out.at[pl.ds(pl.multiple_of(g0 * 16, 8), GPW * 16)])


_sc_agg = pl.kernel(
    _sc_body,
    out_type=[
        jax.ShapeDtypeStruct((N_GROUPS * 2 * H,), jnp.float32),
        jax.ShapeDtypeStruct((N_GROUPS * 16,), jnp.float32),
        jax.ShapeDtypeStruct((B, H), jnp.float32),
    ],
    mesh=plsc.VectorSubcoreMesh(core_axis_name="c", subcore_axis_name="s"),
    compiler_params=pltpu.CompilerParams(needs_layout_passes=False),
    scratch_types=[
        pltpu.VMEM((4, K), jnp.int32),
        pltpu.VMEM((4, K), jnp.int32),
        pltpu.VMEM((K, H), jnp.float32),
        pltpu.VMEM((K, H), jnp.float32),
        pltpu.VMEM((K, H), jnp.float32),
        pltpu.VMEM((K, H), jnp.float32),
        pltpu.VMEM((ACC_N,), jnp.float32),
        pltpu.VMEM((CNT_N,), jnp.float32),
        pltpu.VMEM((16,), jnp.int32),
        pltpu.VMEM((SROWS,), jnp.int32),
        pltpu.VMEM((SROWS, H), jnp.float32),
        pltpu.SemaphoreType.DMA,
        pltpu.SemaphoreType.DMA,
        pltpu.SemaphoreType.DMA,
        pltpu.SemaphoreType.DMA,
    ],
)

GBLK = 640


def _tc_body(sums_ref, cnt_ref, sent_ref, lnr_ref, e_ref, wt_ref, b_ref,
             out_ref):
    cnt = jnp.maximum(jnp.sum(cnt_ref[...], axis=1, keepdims=True), 1.0)
    mean = sums_ref[...] / cnt
    h = jnp.dot(mean, wt_ref[...], preferred_element_type=jnp.float32)
    h = jnp.maximum(h + b_ref[...], 0.0)
    s = jnp.dot(e_ref[...], sent_ref[...], preferred_element_type=jnp.float32)
    tmod = lax.rem(lax.broadcasted_iota(jnp.int32, (GBLK, 1), 0),
                   jnp.int32(SEQ_LEN))
    mask = (tmod.astype(jnp.float32) < lnr_ref[...]).astype(jnp.float32)
    out_ref[...] = jnp.concatenate([h, s], axis=1) * mask


_E = np.repeat(np.eye(GBLK // SEQ_LEN, dtype=np.float32), SEQ_LEN, axis=0)

_tc_pack = pl.pallas_call(
    _tc_body,
    grid=(N_GROUPS // GBLK,),
    in_specs=[
        pl.BlockSpec((GBLK, 2 * H), lambda i: (i, 0)),
        pl.BlockSpec((GBLK, 16), lambda i: (i, 0)),
        pl.BlockSpec((GBLK // SEQ_LEN, H), lambda i: (i, 0)),
        pl.BlockSpec((GBLK, 1), lambda i: (i, 0)),
        pl.BlockSpec((GBLK, GBLK // SEQ_LEN), lambda i: (0, 0)),
        pl.BlockSpec((2 * H, 2 * H), lambda i: (0, 0)),
        pl.BlockSpec((1, 2 * H), lambda i: (0, 0)),
    ],
    out_specs=pl.BlockSpec((GBLK, 3 * H), lambda i: (i, 0)),
    out_shape=jax.ShapeDtypeStruct((N_GROUPS, 3 * H), jnp.float32),
)


def kernel(obj_idx, rel_idx, rel_weights, segment_ids, len_non_zero, s_idx,
           ent_embeds, rel_embeds, W, b):
    obj_idx = obj_idx.astype(jnp.int32)
    rel_idx = rel_idx.astype(jnp.int32)
    seg = segment_ids.astype(jnp.int32)
    s_idx = s_idx.astype(jnp.int32)
    w = rel_weights.astype(jnp.float32)

    zpad_i = jnp.zeros((K,), jnp.int32)
    pidx = jnp.stack([
        jnp.concatenate([obj_idx, zpad_i]),
        jnp.concatenate([rel_idx, zpad_i]),
        jnp.concatenate([seg, jnp.full((K,), SEG_PAD, jnp.int32)]),
        jnp.concatenate([lax.bitcast_convert_type(w, jnp.int32), zpad_i]),
    ])

    bounds = jnp.searchsorted(seg, jnp.arange(NW + 1, dtype=jnp.int32) * GPW)
    starts = ((bounds[:NW] // K) * K).astype(jnp.int32)
    ends = bounds[1:].astype(jnp.int32)

    sums, cnt16, sent = _sc_agg(pidx, starts, ends, s_idx,
                                jnp.zeros((ACC_N,), jnp.float32),
                                ent_embeds, rel_embeds)
    sums = sums.reshape(N_GROUPS, 2 * H)
    cnt16 = cnt16.reshape(N_GROUPS, 16)
    lnr = jnp.repeat(jnp.maximum(len_non_zero, 1).astype(jnp.float32),
                     SEQ_LEN)[:, None]
    out = _tc_pack(sums, cnt16, sent, lnr, jnp.asarray(_E), W.T, b[None, :])
    return out.reshape(B, SEQ_LEN, 3 * H)


# speedup vs baseline: 2.7767x; 1.4753x over previous
import functools

import jax
import jax.numpy as jnp
import numpy as np
from jax import lax
from jax.experimental import pallas as pl
from jax.experimental.pallas import tpu as pltpu
from jax.experimental.pallas import tpu_sc as plsc

T = 32768
N_GROUPS = 5120
B = 512
SEQ_LEN = 10
H = 128
HC = H // 16

NC, NS = 2, 16
NW = NC * NS
GPW = N_GROUPS // NW
GSC = N_GROUPS // NC
K = 128
KC = K // 16
SROWS = B // NW
SEG_PAD = 1 << 20

CNT_N = (GPW + 1) * 16
ASH_ROWS = GSC + 16
ZROWS = ASH_ROWS // NS


def _sc_body(pidx_hbm, starts_hbm, ends_hbm, scb_hbm, sidx_hbm, zeros_hbm,
             ent_hbm, rele_hbm,
             srel_out, sent_o_out, cnt_out, sent_out,
             pidx0, pidx1, wrel0, wrel1, went0, went1, lidx0, lidx1,
             cnt_v, b_v, sidx_v, srow, arel_sh, aent_sh,
             sem_gr0, sem_gr1, sem_ge0, sem_ge1,
             sem_sr0, sem_sr1, sem_se0, sem_se1):
    cid = lax.axis_index("c")
    sid = lax.axis_index("s")
    wid = cid * NS + sid
    g0 = wid * GPW
    lanes = lax.iota(jnp.int32, 16)
    iota16 = lanes

    z0 = pl.multiple_of(sid * ZROWS, 8)
    pltpu.sync_copy(zeros_hbm, arel_sh.at[pl.ds(z0, ZROWS)])
    pltpu.sync_copy(zeros_hbm, aent_sh.at[pl.ds(z0, ZROWS)])
    zero16 = jnp.zeros((16,), jnp.float32)

    def zcnt(r, carry):
        cnt_v[pl.ds(r * 16, 16)] = zero16
        return carry

    lax.fori_loop(0, CNT_N // 16, zcnt, 0)

    srow0 = pl.multiple_of(wid * SROWS, 8)
    pltpu.sync_copy(sidx_hbm.at[pl.ds(srow0, SROWS)], sidx_v)
    pltpu.sync_copy(ent_hbm.at[sidx_v], srow)
    pltpu.sync_copy(srow, sent_out.at[pl.ds(srow0, SROWS)])

    pltpu.sync_copy(starts_hbm.at[pl.ds(cid * 16, 16)], b_v)
    astart = jnp.max(jnp.where(lanes == sid, b_v[...], 0))
    pltpu.sync_copy(ends_hbm.at[pl.ds(cid * 16, 16)], b_v)
    aend = jnp.max(jnp.where(lanes == sid, b_v[...], 0))
    nblk_a = (aend - astart + K - 1) // K

    def cnt_blk(bi, carry):
        base = pl.multiple_of(astart + bi * K, K)
        pltpu.sync_copy(pidx_hbm.at[:, pl.ds(base, K)], pidx0)

        def chunk(ci, c2):
            sc = pidx0[2, pl.ds(ci * 16, 16)]
            ls = sc - g0
            valid = (ls >= 0) & (ls < GPW)
            lsc = jnp.where(valid, ls, GPW)
            validf = jnp.where(valid, 1.0, 0.0)
            plsc.addupdate_scatter(cnt_v, [lsc * 16 + iota16], validf)
            return c2

        lax.fori_loop(0, KC, chunk, 0)
        return carry

    lax.fori_loop(0, nblk_a, cnt_blk, 0)
    pltpu.sync_copy(cnt_v.at[pl.ds(0, GPW * 16)],
                    cnt_out.at[pl.ds(pl.multiple_of(g0 * 16, 8), GPW * 16)])

    cg0 = cid * GSC
    pltpu.sync_copy(scb_hbm.at[pl.ds(0, 16)], b_v)
    cstart = jnp.max(jnp.where(lanes == cid * 2, b_v[...], 0))
    cend = jnp.max(jnp.where(lanes == cid * 2 + 1, b_v[...], 0))
    nblk_b = (cend - cstart + K - 1) // K
    nmy = (nblk_b - sid + NS - 1) // NS

    plsc.subcore_barrier()

    def issue(k, pidx_v, wrel, went, sem_gr, sem_ge, sem_sr, sem_se, lidx):
        @pl.when(k >= 2)
        def _():
            pltpu.make_async_copy(wrel, arel_sh.at[lidx], sem_sr).wait()
            pltpu.make_async_copy(went, aent_sh.at[lidx], sem_se).wait()

        base = pl.multiple_of(cstart + (sid + k * NS) * K, K)
        pltpu.sync_copy(pidx_hbm.at[:, pl.ds(base, K)], pidx_v)
        pltpu.async_copy(rele_hbm.at[pidx_v.at[1]], wrel, sem_gr)
        pltpu.async_copy(ent_hbm.at[pidx_v.at[0]], went, sem_ge)

    def compute(pidx_v, wrel, went, sem_gr, sem_ge, sem_sr, sem_se, lidx):
        pltpu.make_async_copy(rele_hbm.at[pidx_v.at[1]], wrel, sem_gr).wait()
        pltpu.make_async_copy(ent_hbm.at[pidx_v.at[0]], went, sem_ge).wait()

        def chunk(ci, c2):
            sc = pidx_v[2, pl.ds(ci * 16, 16)]
            wc = plsc.bitcast(pidx_v[3, pl.ds(ci * 16, 16)], jnp.float32)
            ls = sc - cg0
            valid = (ls >= 0) & (ls < GSC)
            lsc = jnp.where(valid, ls, GSC)
            wz = jnp.where(valid, wc, 0.0)
            lidx[pl.ds(ci * 16, 16)] = lsc
            for j in range(16):
                jf = jnp.full((16,), j, jnp.int32)
                wj = jnp.take(wz, jf)
                r = ci * 16 + j
                for c in range(HC):
                    wrel[r, pl.ds(c * 16, 16)] = wrel[r, pl.ds(c * 16, 16)] * wj
                for c in range(HC):
                    went[r, pl.ds(c * 16, 16)] = went[r, pl.ds(c * 16, 16)] * wj
            return c2

        lax.fori_loop(0, KC, chunk, 0)
        pltpu.async_copy(wrel, arel_sh.at[lidx], sem_sr, add=True)
        pltpu.async_copy(went, aent_sh.at[lidx], sem_se, add=True)

    @pl.when(nmy > 0)
    def _():
        issue(0, pidx0, wrel0, went0, sem_gr0, sem_ge0, sem_sr0, sem_se0,
              lidx0)

    def blk(k, carry):
        nxt = k + 1

        @pl.when((nxt < nmy) & (nxt % 2 == 0))
        def _():
            issue(nxt, pidx0, wrel0, went0, sem_gr0, sem_ge0, sem_sr0,
                  sem_se0, lidx0)

        @pl.when((nxt < nmy) & (nxt % 2 == 1))
        def _():
            issue(nxt, pidx1, wrel1, went1, sem_gr1, sem_ge1, sem_sr1,
                  sem_se1, lidx1)

        @pl.when(k % 2 == 0)
        def _():
            compute(pidx0, wrel0, went0, sem_gr0, sem_ge0, sem_sr0, sem_se0,
                    lidx0)

        @pl.when(k % 2 == 1)
        def _():
            compute(pidx1, wrel1, went1, sem_gr1, sem_ge1, sem_sr1, sem_se1,
                    lidx1)

        return carry

    lax.fori_loop(0, nmy, blk, 0)

    @pl.when(nmy >= 1)
    def _():
        @pl.when(nmy % 2 == 1)
        def _():
            pltpu.make_async_copy(wrel0, arel_sh.at[lidx0], sem_sr0).wait()
            pltpu.make_async_copy(went0, aent_sh.at[lidx0], sem_se0).wait()

        @pl.when(nmy % 2 == 0)
        def _():
            pltpu.make_async_copy(wrel1, arel_sh.at[lidx1], sem_sr1).wait()
            pltpu.make_async_copy(went1, aent_sh.at[lidx1], sem_se1).wait()

    @pl.when(nmy >= 2)
    def _():
        @pl.when(nmy % 2 == 0)
        def _():
            pltpu.make_async_copy(wrel0, arel_sh.at[lidx0], sem_sr0).wait()
            pltpu.make_async_copy(went0, aent_sh.at[lidx0], sem_se0).wait()

        @pl.when(nmy % 2 == 1)
        def _():
            pltpu.make_async_copy(wrel1, arel_sh.at[lidx1], sem_sr1).wait()
            pltpu.make_async_copy(went1, aent_sh.at[lidx1], sem_se1).wait()

    plsc.subcore_barrier()

    r0 = pl.multiple_of(sid * (GSC // NS), 8)
    o0 = pl.multiple_of(cg0 + sid * (GSC // NS), 8)
    pltpu.sync_copy(arel_sh.at[pl.ds(r0, GSC // NS)],
                    srel_out.at[pl.ds(o0, GSC // NS)])
    pltpu.sync_copy(aent_sh.at[pl.ds(r0, GSC // NS)],
                    sent_o_out.at[pl.ds(o0, GSC // NS)])


_sc_agg = pl.kernel(
    _sc_body,
    out_type=[
        jax.ShapeDtypeStruct((N_GROUPS, H), jnp.float32),
        jax.ShapeDtypeStruct((N_GROUPS, H), jnp.float32),
        jax.ShapeDtypeStruct((N_GROUPS * 16,), jnp.float32),
        jax.ShapeDtypeStruct((B, H), jnp.float32),
    ],
    mesh=plsc.VectorSubcoreMesh(core_axis_name="c", subcore_axis_name="s"),
    compiler_params=pltpu.CompilerParams(needs_layout_passes=False),
    scratch_types=[
        pltpu.VMEM((4, K), jnp.int32),
        pltpu.VMEM((4, K), jnp.int32),
        pltpu.VMEM((K, H), jnp.float32),
        pltpu.VMEM((K, H), jnp.float32),
        pltpu.VMEM((K, H), jnp.float32),
        pltpu.VMEM((K, H), jnp.float32),
        pltpu.VMEM((K,), jnp.int32),
        pltpu.VMEM((K,), jnp.int32),
        pltpu.VMEM((CNT_N,), jnp.float32),
        pltpu.VMEM((16,), jnp.int32),
        pltpu.VMEM((SROWS,), jnp.int32),
        pltpu.VMEM((SROWS, H), jnp.float32),
        pltpu.VMEM_SHARED((ASH_ROWS, H), jnp.float32),
        pltpu.VMEM_SHARED((ASH_ROWS, H), jnp.float32),
        pltpu.SemaphoreType.DMA,
        pltpu.SemaphoreType.DMA,
        pltpu.SemaphoreType.DMA,
        pltpu.SemaphoreType.DMA,
        pltpu.SemaphoreType.DMA,
        pltpu.SemaphoreType.DMA,
        pltpu.SemaphoreType.DMA,
        pltpu.SemaphoreType.DMA,
    ],
)

GBLK = 640


def _tc_body(srel_ref, sent_s_ref, cnt_ref, sent_ref, lnr_ref, e_ref,
             wt_ref, b_ref, out_ref):
    cnt = jnp.maximum(jnp.sum(cnt_ref[...], axis=1, keepdims=True), 1.0)
    mean = jnp.concatenate([srel_ref[...], sent_s_ref[...]], axis=1) / cnt
    h = jnp.dot(mean, wt_ref[...], preferred_element_type=jnp.float32)
    h = jnp.maximum(h + b_ref[...], 0.0)
    s = jnp.dot(e_ref[...], sent_ref[...], preferred_element_type=jnp.float32)
    tmod = lax.rem(lax.broadcasted_iota(jnp.int32, (GBLK, 1), 0),
                   jnp.int32(SEQ_LEN))
    mask = (tmod.astype(jnp.float32) < lnr_ref[...]).astype(jnp.float32)
    out_ref[...] = jnp.concatenate([h, s], axis=1) * mask


_E = np.repeat(np.eye(GBLK // SEQ_LEN, dtype=np.float32), SEQ_LEN, axis=0)

_tc_pack = pl.pallas_call(
    _tc_body,
    grid=(N_GROUPS // GBLK,),
    in_specs=[
        pl.BlockSpec((GBLK, H), lambda i: (i, 0)),
        pl.BlockSpec((GBLK, H), lambda i: (i, 0)),
        pl.BlockSpec((GBLK, 16), lambda i: (i, 0)),
        pl.BlockSpec((GBLK // SEQ_LEN, H), lambda i: (i, 0)),
        pl.BlockSpec((GBLK, 1), lambda i: (i, 0)),
        pl.BlockSpec((GBLK, GBLK // SEQ_LEN), lambda i: (0, 0)),
        pl.BlockSpec((2 * H, 2 * H), lambda i: (0, 0)),
        pl.BlockSpec((1, 2 * H), lambda i: (0, 0)),
    ],
    out_specs=pl.BlockSpec((GBLK, 3 * H), lambda i: (i, 0)),
    out_shape=jax.ShapeDtypeStruct((N_GROUPS, 3 * H), jnp.float32),
)


def kernel(obj_idx, rel_idx, rel_weights, segment_ids, len_non_zero, s_idx,
           ent_embeds, rel_embeds, W, b):
    obj_idx = obj_idx.astype(jnp.int32)
    rel_idx = rel_idx.astype(jnp.int32)
    seg = segment_ids.astype(jnp.int32)
    s_idx = s_idx.astype(jnp.int32)
    w = rel_weights.astype(jnp.float32)

    zpad_i = jnp.zeros((K,), jnp.int32)
    pidx = jnp.stack([
        jnp.concatenate([obj_idx, zpad_i]),
        jnp.concatenate([rel_idx, zpad_i]),
        jnp.concatenate([seg, jnp.full((K,), SEG_PAD, jnp.int32)]),
        jnp.concatenate([lax.bitcast_convert_type(w, jnp.int32), zpad_i]),
    ])

    bounds = jnp.searchsorted(seg, jnp.arange(NW + 1, dtype=jnp.int32) * GPW)
    starts = ((bounds[:NW] // K) * K).astype(jnp.int32)
    ends = bounds[1:].astype(jnp.int32)
    scb = jnp.zeros((16,), jnp.int32)
    scb = scb.at[0].set((bounds[0] // K) * K).at[1].set(bounds[NS])
    scb = scb.at[2].set((bounds[NS] // K) * K).at[3].set(bounds[NW])

    srel, sent_s, cnt16, sent = _sc_agg(
        pidx, starts, ends, scb, s_idx,
        jnp.zeros((ZROWS, H), jnp.float32), ent_embeds, rel_embeds)
    cnt16 = cnt16.reshape(N_GROUPS, 16)
    lnr = jnp.repeat(jnp.maximum(len_non_zero, 1).astype(jnp.float32),
                     SEQ_LEN)[:, None]
    out = _tc_pack(srel, sent_s, cnt16, sent, lnr, jnp.asarray(_E),
                   W.T, b[None, :])
    return out.reshape(B, SEQ_LEN, 3 * H)


# speedup vs baseline: 3.3761x; 1.2158x over previous
import functools

import jax
import jax.numpy as jnp
import numpy as np
from jax import lax
from jax.experimental import pallas as pl
from jax.experimental.pallas import tpu as pltpu
from jax.experimental.pallas import tpu_sc as plsc

T = 32768
N_GROUPS = 5120
B = 512
SEQ_LEN = 10
H = 128
HC = H // 16

NC, NS = 2, 16
NW = NC * NS
GPW = N_GROUPS // NW
GSC = N_GROUPS // NC
K = 128
KC = K // 16
SROWS = B // NW
SEG_PAD = 1 << 20

CNT_N = (GPW + 1) * 16
ASH_ROWS = GSC + 16
ZROWS = ASH_ROWS // NS


def _sc_body(pidx_hbm, starts_hbm, ends_hbm, scb_hbm, sidx_hbm, zeros_hbm,
             ent_hbm, rele_hbm,
             srel_out, sent_o_out, cnt_out, sent_out,
             pidx0, pidx1, wrel0, wrel1, went0, went1, lidx0, lidx1,
             cnt_v, b_v, sidx_v, srow, arel_sh, aent_sh,
             sem_gr0, sem_gr1, sem_ge0, sem_ge1,
             sem_sr0, sem_sr1, sem_se0, sem_se1, sem_a0, sem_a1, sem_z):
    cid = lax.axis_index("c")
    sid = lax.axis_index("s")
    wid = cid * NS + sid
    g0 = wid * GPW
    lanes = lax.iota(jnp.int32, 16)
    iota16 = lanes

    z0 = pl.multiple_of(sid * ZROWS, 8)
    pltpu.async_copy(zeros_hbm, arel_sh.at[pl.ds(z0, ZROWS)], sem_z)
    pltpu.async_copy(zeros_hbm, aent_sh.at[pl.ds(z0, ZROWS)], sem_z)
    zero16 = jnp.zeros((16,), jnp.float32)

    def zcnt(r, carry):
        cnt_v[pl.ds(r * 16, 16)] = zero16
        return carry

    lax.fori_loop(0, CNT_N // 16, zcnt, 0)

    pltpu.sync_copy(starts_hbm.at[pl.ds(cid * 16, 16)], b_v)
    astart = jnp.max(jnp.where(lanes == sid, b_v[...], 0))
    pltpu.sync_copy(ends_hbm.at[pl.ds(cid * 16, 16)], b_v)
    aend = jnp.max(jnp.where(lanes == sid, b_v[...], 0))
    nblk_a = (aend - astart + K - 1) // K

    cg0 = cid * GSC
    pltpu.sync_copy(scb_hbm.at[pl.ds(0, 16)], b_v)
    cstart = jnp.max(jnp.where(lanes == cid * 2, b_v[...], 0))
    cend = jnp.max(jnp.where(lanes == cid * 2 + 1, b_v[...], 0))
    nblk_b = (cend - cstart + K - 1) // K
    nmy = (nblk_b - sid + NS - 1) // NS

    pltpu.make_async_copy(zeros_hbm, arel_sh.at[pl.ds(z0, ZROWS)],
                          sem_z).wait()
    pltpu.make_async_copy(zeros_hbm, aent_sh.at[pl.ds(z0, ZROWS)],
                          sem_z).wait()
    plsc.subcore_barrier()

    def issue(k, pidx_v, wrel, went, sem_gr, sem_ge, sem_sr, sem_se, lidx):
        @pl.when(k >= 2)
        def _():
            pltpu.make_async_copy(wrel, arel_sh.at[lidx], sem_sr).wait()
            pltpu.make_async_copy(went, aent_sh.at[lidx], sem_se).wait()

        base = pl.multiple_of(cstart + (sid + k * NS) * K, K)
        pltpu.sync_copy(pidx_hbm.at[:, pl.ds(base, K)], pidx_v)
        pltpu.async_copy(rele_hbm.at[pidx_v.at[1]], wrel, sem_gr)
        pltpu.async_copy(ent_hbm.at[pidx_v.at[0]], went, sem_ge)

    def compute(pidx_v, wrel, went, sem_gr, sem_ge, sem_sr, sem_se, lidx):
        pltpu.make_async_copy(rele_hbm.at[pidx_v.at[1]], wrel, sem_gr).wait()
        pltpu.make_async_copy(ent_hbm.at[pidx_v.at[0]], went, sem_ge).wait()

        def chunk(ci, c2):
            sc = pidx_v[2, pl.ds(ci * 16, 16)]
            wc = plsc.bitcast(pidx_v[3, pl.ds(ci * 16, 16)], jnp.float32)
            ls = sc - cg0
            valid = (ls >= 0) & (ls < GSC)
            lsc = jnp.where(valid, ls, GSC)
            wz = jnp.where(valid, wc, 0.0)
            lidx[pl.ds(ci * 16, 16)] = lsc
            for j in range(16):
                jf = jnp.full((16,), j, jnp.int32)
                wj = jnp.take(wz, jf)
                r = ci * 16 + j
                for c in range(HC):
                    wrel[r, pl.ds(c * 16, 16)] = wrel[r, pl.ds(c * 16, 16)] * wj
                for c in range(HC):
                    went[r, pl.ds(c * 16, 16)] = went[r, pl.ds(c * 16, 16)] * wj
            return c2

        lax.fori_loop(0, KC, chunk, 0)
        pltpu.async_copy(wrel, arel_sh.at[lidx], sem_sr, add=True)
        pltpu.async_copy(went, aent_sh.at[lidx], sem_se, add=True)

    @pl.when(nmy > 0)
    def _():
        issue(0, pidx0, wrel0, went0, sem_gr0, sem_ge0, sem_sr0, sem_se0,
              lidx0)

    srow0 = pl.multiple_of(wid * SROWS, 8)
    pltpu.sync_copy(sidx_hbm.at[pl.ds(srow0, SROWS)], sidx_v)
    pltpu.sync_copy(ent_hbm.at[sidx_v], srow)
    pltpu.sync_copy(srow, sent_out.at[pl.ds(srow0, SROWS)])

    def cissue(bi, segb, sem):
        base = pl.multiple_of(astart + bi * K, K)
        pltpu.async_copy(pidx_hbm.at[2, pl.ds(base, K)], segb, sem)

    def ccompute(segb, sem):
        pltpu.make_async_copy(pidx_hbm.at[2, pl.ds(0, K)], segb, sem).wait()

        def chunk(ci, c2):
            sc = segb[pl.ds(ci * 16, 16)]
            ls = sc - g0
            valid = (ls >= 0) & (ls < GPW)
            lsc = jnp.where(valid, ls, GPW)
            validf = jnp.where(valid, 1.0, 0.0)
            plsc.addupdate_scatter(cnt_v, [lsc * 16 + iota16], validf)
            return c2

        lax.fori_loop(0, KC, chunk, 0)

    @pl.when(nblk_a > 0)
    def _():
        cissue(0, lidx0, sem_a0)

    def cnt_blk(bi, carry):
        nxt = bi + 1

        @pl.when((nxt < nblk_a) & (nxt % 2 == 0))
        def _():
            cissue(nxt, lidx0, sem_a0)

        @pl.when((nxt < nblk_a) & (nxt % 2 == 1))
        def _():
            cissue(nxt, lidx1, sem_a1)

        @pl.when(bi % 2 == 0)
        def _():
            ccompute(lidx0, sem_a0)

        @pl.when(bi % 2 == 1)
        def _():
            ccompute(lidx1, sem_a1)

        return carry

    lax.fori_loop(0, nblk_a, cnt_blk, 0)
    pltpu.sync_copy(cnt_v.at[pl.ds(0, GPW * 16)],
                    cnt_out.at[pl.ds(pl.multiple_of(g0 * 16, 8), GPW * 16)])

    def blk(k, carry):
        nxt = k + 1

        @pl.when((nxt < nmy) & (nxt % 2 == 0))
        def _():
            issue(nxt, pidx0, wrel0, went0, sem_gr0, sem_ge0, sem_sr0,
                  sem_se0, lidx0)

        @pl.when((nxt < nmy) & (nxt % 2 == 1))
        def _():
            issue(nxt, pidx1, wrel1, went1, sem_gr1, sem_ge1, sem_sr1,
                  sem_se1, lidx1)

        @pl.when(k % 2 == 0)
        def _():
            compute(pidx0, wrel0, went0, sem_gr0, sem_ge0, sem_sr0, sem_se0,
                    lidx0)

        @pl.when(k % 2 == 1)
        def _():
            compute(pidx1, wrel1, went1, sem_gr1, sem_ge1, sem_sr1, sem_se1,
                    lidx1)

        return carry

    lax.fori_loop(0, nmy, blk, 0)

    @pl.when(nmy >= 1)
    def _():
        @pl.when(nmy % 2 == 1)
        def _():
            pltpu.make_async_copy(wrel0, arel_sh.at[lidx0], sem_sr0).wait()
            pltpu.make_async_copy(went0, aent_sh.at[lidx0], sem_se0).wait()

        @pl.when(nmy % 2 == 0)
        def _():
            pltpu.make_async_copy(wrel1, arel_sh.at[lidx1], sem_sr1).wait()
            pltpu.make_async_copy(went1, aent_sh.at[lidx1], sem_se1).wait()

    @pl.when(nmy >= 2)
    def _():
        @pl.when(nmy % 2 == 0)
        def _():
            pltpu.make_async_copy(wrel0, arel_sh.at[lidx0], sem_sr0).wait()
            pltpu.make_async_copy(went0, aent_sh.at[lidx0], sem_se0).wait()

        @pl.when(nmy % 2 == 1)
        def _():
            pltpu.make_async_copy(wrel1, arel_sh.at[lidx1], sem_sr1).wait()
            pltpu.make_async_copy(went1, aent_sh.at[lidx1], sem_se1).wait()

    plsc.subcore_barrier()

    r0 = pl.multiple_of(sid * (GSC // NS), 8)
    o0 = pl.multiple_of(cg0 + sid * (GSC // NS), 8)
    pltpu.sync_copy(arel_sh.at[pl.ds(r0, GSC // NS)],
                    srel_out.at[pl.ds(o0, GSC // NS)])
    pltpu.sync_copy(aent_sh.at[pl.ds(r0, GSC // NS)],
                    sent_o_out.at[pl.ds(o0, GSC // NS)])


_sc_agg = pl.kernel(
    _sc_body,
    out_type=[
        jax.ShapeDtypeStruct((N_GROUPS, H), jnp.float32),
        jax.ShapeDtypeStruct((N_GROUPS, H), jnp.float32),
        jax.ShapeDtypeStruct((N_GROUPS * 16,), jnp.float32),
        jax.ShapeDtypeStruct((B, H), jnp.float32),
    ],
    mesh=plsc.VectorSubcoreMesh(core_axis_name="c", subcore_axis_name="s"),
    compiler_params=pltpu.CompilerParams(needs_layout_passes=False),
    scratch_types=[
        pltpu.VMEM((4, K), jnp.int32),
        pltpu.VMEM((4, K), jnp.int32),
        pltpu.VMEM((K, H), jnp.float32),
        pltpu.VMEM((K, H), jnp.float32),
        pltpu.VMEM((K, H), jnp.float32),
        pltpu.VMEM((K, H), jnp.float32),
        pltpu.VMEM((K,), jnp.int32),
        pltpu.VMEM((K,), jnp.int32),
        pltpu.VMEM((CNT_N,), jnp.float32),
        pltpu.VMEM((16,), jnp.int32),
        pltpu.VMEM((SROWS,), jnp.int32),
        pltpu.VMEM((SROWS, H), jnp.float32),
        pltpu.VMEM_SHARED((ASH_ROWS, H), jnp.float32),
        pltpu.VMEM_SHARED((ASH_ROWS, H), jnp.float32),
        pltpu.SemaphoreType.DMA,
        pltpu.SemaphoreType.DMA,
        pltpu.SemaphoreType.DMA,
        pltpu.SemaphoreType.DMA,
        pltpu.SemaphoreType.DMA,
        pltpu.SemaphoreType.DMA,
        pltpu.SemaphoreType.DMA,
        pltpu.SemaphoreType.DMA,
        pltpu.SemaphoreType.DMA,
        pltpu.SemaphoreType.DMA,
        pltpu.SemaphoreType.DMA,
    ],
)

GBLK = 640


def _tc_body(srel_ref, sent_s_ref, cnt_ref, sent_ref, lnr_ref, e_ref,
             wt_ref, b_ref, out_ref):
    cnt = jnp.maximum(jnp.sum(cnt_ref[...], axis=1, keepdims=True), 1.0)
    mean = jnp.concatenate([srel_ref[...], sent_s_ref[...]], axis=1) / cnt
    h = jnp.dot(mean, wt_ref[...], preferred_element_type=jnp.float32)
    h = jnp.maximum(h + b_ref[...], 0.0)
    s = jnp.dot(e_ref[...], sent_ref[...], preferred_element_type=jnp.float32)
    tmod = lax.rem(lax.broadcasted_iota(jnp.int32, (GBLK, 1), 0),
                   jnp.int32(SEQ_LEN))
    mask = (tmod.astype(jnp.float32) < lnr_ref[...]).astype(jnp.float32)
    out_ref[...] = jnp.concatenate([h, s], axis=1) * mask


_E = np.repeat(np.eye(GBLK // SEQ_LEN, dtype=np.float32), SEQ_LEN, axis=0)

_tc_pack = pl.pallas_call(
    _tc_body,
    grid=(N_GROUPS // GBLK,),
    in_specs=[
        pl.BlockSpec((GBLK, H), lambda i: (i, 0)),
        pl.BlockSpec((GBLK, H), lambda i: (i, 0)),
        pl.BlockSpec((GBLK, 16), lambda i: (i, 0)),
        pl.BlockSpec((GBLK // SEQ_LEN, H), lambda i: (i, 0)),
        pl.BlockSpec((GBLK, 1), lambda i: (i, 0)),
        pl.BlockSpec((GBLK, GBLK // SEQ_LEN), lambda i: (0, 0)),
        pl.BlockSpec((2 * H, 2 * H), lambda i: (0, 0)),
        pl.BlockSpec((1, 2 * H), lambda i: (0, 0)),
    ],
    out_specs=pl.BlockSpec((GBLK, 3 * H), lambda i: (i, 0)),
    out_shape=jax.ShapeDtypeStruct((N_GROUPS, 3 * H), jnp.float32),
)


def kernel(obj_idx, rel_idx, rel_weights, segment_ids, len_non_zero, s_idx,
           ent_embeds, rel_embeds, W, b):
    obj_idx = obj_idx.astype(jnp.int32)
    rel_idx = rel_idx.astype(jnp.int32)
    seg = segment_ids.astype(jnp.int32)
    s_idx = s_idx.astype(jnp.int32)
    w = rel_weights.astype(jnp.float32)

    zpad_i = jnp.zeros((K,), jnp.int32)
    pidx = jnp.stack([
        jnp.concatenate([obj_idx, zpad_i]),
        jnp.concatenate([rel_idx, zpad_i]),
        jnp.concatenate([seg, jnp.full((K,), SEG_PAD, jnp.int32)]),
        jnp.concatenate([lax.bitcast_convert_type(w, jnp.int32), zpad_i]),
    ])

    bounds = jnp.searchsorted(seg, jnp.arange(NW + 1, dtype=jnp.int32) * GPW,
                              method="compare_all")
    starts = ((bounds[:NW] // K) * K).astype(jnp.int32)
    ends = bounds[1:].astype(jnp.int32)
    scb = jnp.zeros((16,), jnp.int32)
    scb = scb.at[0].set((bounds[0] // K) * K).at[1].set(bounds[NS])
    scb = scb.at[2].set((bounds[NS] // K) * K).at[3].set(bounds[NW])

    srel, sent_s, cnt16, sent = _sc_agg(
        pidx, starts, ends, scb, s_idx,
        jnp.zeros((ZROWS, H), jnp.float32), ent_embeds, rel_embeds)
    cnt16 = cnt16.reshape(N_GROUPS, 16)
    lnr = jnp.repeat(jnp.maximum(len_non_zero, 1).astype(jnp.float32),
                     SEQ_LEN)[:, None]
    out = _tc_pack(srel, sent_s, cnt16, sent, lnr, jnp.asarray(_E),
                   W.T, b[None, :])
    return out.reshape(B, SEQ_LEN, 3 * H)


# speedup vs baseline: 3.5538x; 1.0526x over previous
import functools

import jax
import jax.numpy as jnp
import numpy as np
from jax import lax
from jax.experimental import pallas as pl
from jax.experimental.pallas import tpu as pltpu
from jax.experimental.pallas import tpu_sc as plsc

T = 32768
N_GROUPS = 5120
B = 512
SEQ_LEN = 10
H = 128
HC = H // 16

NC, NS = 2, 16
NW = NC * NS
GPW = N_GROUPS // NW
GSC = N_GROUPS // NC
K = 128
KC = K // 16
SROWS = B // NW
SEG_PAD = 1 << 20

CNT_N = (GPW + 1) * 16
ASH_ROWS = GSC + 16
ZROWS = ASH_ROWS // NS


def _sc_body(pidx_hbm, starts_hbm, ends_hbm, scb_hbm, sidx_hbm, zeros_hbm,
             tperm_hbm, ent_hbm, rele_hbm,
             srel_out, sent_o_out, cnt_out, sent_out,
             pidx0, pidx1, wrel0, wrel1, went0, went1, lidx0, lidx1,
             cnt_v, b_v, sidx_v, srow, i128, i32b, arel_sh, aent_sh,
             sem_gr0, sem_gr1, sem_ge0, sem_ge1,
             sem_sr0, sem_sr1, sem_se0, sem_se1, sem_a0, sem_a1, sem_z):
    cid = lax.axis_index("c")
    sid = lax.axis_index("s")
    wid = cid * NS + sid
    g0 = wid * GPW
    lanes = lax.iota(jnp.int32, 16)
    iota16 = lanes

    z0 = pl.multiple_of(sid * ZROWS, 8)
    pltpu.async_copy(zeros_hbm, arel_sh.at[pl.ds(z0, ZROWS)], sem_z)
    pltpu.async_copy(zeros_hbm, aent_sh.at[pl.ds(z0, ZROWS)], sem_z)
    zero16 = jnp.zeros((16,), jnp.float32)

    def zcnt(r, carry):
        cnt_v[pl.ds(r * 16, 16)] = zero16
        return carry

    lax.fori_loop(0, CNT_N // 16, zcnt, 0)

    pltpu.sync_copy(starts_hbm.at[pl.ds(cid * 16, 16)], b_v)
    astart = jnp.max(jnp.where(lanes == sid, b_v[...], 0))
    pltpu.sync_copy(ends_hbm.at[pl.ds(cid * 16, 16)], b_v)
    aend = jnp.max(jnp.where(lanes == sid, b_v[...], 0))
    nblk_a = (aend - astart + K - 1) // K

    cg0 = cid * GSC
    pltpu.sync_copy(scb_hbm.at[pl.ds(0, 16)], b_v)
    cstart = jnp.max(jnp.where(lanes == cid * 2, b_v[...], 0))
    cend = jnp.max(jnp.where(lanes == cid * 2 + 1, b_v[...], 0))
    nblk_b = (cend - cstart + K - 1) // K
    nmy = (nblk_b - sid + NS - 1) // NS

    pltpu.make_async_copy(zeros_hbm, arel_sh.at[pl.ds(z0, ZROWS)],
                          sem_z).wait()
    pltpu.make_async_copy(zeros_hbm, aent_sh.at[pl.ds(z0, ZROWS)],
                          sem_z).wait()
    plsc.subcore_barrier()

    def issue(k, pidx_v, wrel, went, sem_gr, sem_ge, sem_sr, sem_se, lidx):
        @pl.when(k >= 2)
        def _():
            pltpu.make_async_copy(wrel, arel_sh.at[lidx], sem_sr).wait()
            pltpu.make_async_copy(went, aent_sh.at[lidx], sem_se).wait()

        base = pl.multiple_of(cstart + (sid + k * NS) * K, K)
        pltpu.sync_copy(pidx_hbm.at[:, pl.ds(base, K)], pidx_v)
        pltpu.async_copy(rele_hbm.at[pidx_v.at[1]], wrel, sem_gr)
        pltpu.async_copy(ent_hbm.at[pidx_v.at[0]], went, sem_ge)

    def compute(pidx_v, wrel, went, sem_gr, sem_ge, sem_sr, sem_se, lidx):
        pltpu.make_async_copy(rele_hbm.at[pidx_v.at[1]], wrel, sem_gr).wait()
        pltpu.make_async_copy(ent_hbm.at[pidx_v.at[0]], went, sem_ge).wait()

        def chunk(ci, c2):
            sc = pidx_v[2, pl.ds(ci * 16, 16)]
            wc = plsc.bitcast(pidx_v[3, pl.ds(ci * 16, 16)], jnp.float32)
            ls = sc - cg0
            valid = (ls >= 0) & (ls < GSC)
            lsc = jnp.where(valid, ls, GSC)
            wz = jnp.where(valid, wc, 0.0)
            lidx[pl.ds(ci * 16, 16)] = lsc
            for j in range(16):
                jf = jnp.full((16,), j, jnp.int32)
                wj = jnp.take(wz, jf)
                r = ci * 16 + j
                for c in range(HC):
                    wrel[r, pl.ds(c * 16, 16)] = wrel[r, pl.ds(c * 16, 16)] * wj
                for c in range(HC):
                    went[r, pl.ds(c * 16, 16)] = went[r, pl.ds(c * 16, 16)] * wj
            return c2

        lax.fori_loop(0, KC, chunk, 0)
        pltpu.async_copy(wrel, arel_sh.at[lidx], sem_sr, add=True)
        pltpu.async_copy(went, aent_sh.at[lidx], sem_se, add=True)

    @pl.when(nmy > 0)
    def _():
        issue(0, pidx0, wrel0, went0, sem_gr0, sem_ge0, sem_sr0, sem_se0,
              lidx0)

    srow0 = pl.multiple_of(wid * SROWS, 8)
    pltpu.sync_copy(sidx_hbm.at[pl.ds(srow0, SROWS)], sidx_v)
    pltpu.sync_copy(ent_hbm.at[sidx_v], srow)
    pltpu.sync_copy(srow, sent_out.at[pl.ds(srow0, SROWS)])

    def cissue(bi, segb, sem):
        base = pl.multiple_of(astart + bi * K, K)
        pltpu.async_copy(pidx_hbm.at[2, pl.ds(base, K)], segb, sem)

    def ccompute(segb, sem):
        pltpu.make_async_copy(pidx_hbm.at[2, pl.ds(0, K)], segb, sem).wait()

        def chunk(ci, c2):
            sc = segb[pl.ds(ci * 16, 16)]
            ls = sc - g0
            valid = (ls >= 0) & (ls < GPW)
            lsc = jnp.where(valid, ls, GPW)
            validf = jnp.where(valid, 1.0, 0.0)
            plsc.addupdate_scatter(cnt_v, [lsc * 16 + iota16], validf)
            return c2

        lax.fori_loop(0, KC, chunk, 0)

    @pl.when(nblk_a > 0)
    def _():
        cissue(0, lidx0, sem_a0)

    def cnt_blk(bi, carry):
        nxt = bi + 1

        @pl.when((nxt < nblk_a) & (nxt % 2 == 0))
        def _():
            cissue(nxt, lidx0, sem_a0)

        @pl.when((nxt < nblk_a) & (nxt % 2 == 1))
        def _():
            cissue(nxt, lidx1, sem_a1)

        @pl.when(bi % 2 == 0)
        def _():
            ccompute(lidx0, sem_a0)

        @pl.when(bi % 2 == 1)
        def _():
            ccompute(lidx1, sem_a1)

        return carry

    lax.fori_loop(0, nblk_a, cnt_blk, 0)
    pltpu.sync_copy(cnt_v.at[pl.ds(0, GPW * 16)],
                    cnt_out.at[pl.ds(pl.multiple_of(g0 * 16, 8), GPW * 16)])

    def blk(k, carry):
        nxt = k + 1

        @pl.when((nxt < nmy) & (nxt % 2 == 0))
        def _():
            issue(nxt, pidx0, wrel0, went0, sem_gr0, sem_ge0, sem_sr0,
                  sem_se0, lidx0)

        @pl.when((nxt < nmy) & (nxt % 2 == 1))
        def _():
            issue(nxt, pidx1, wrel1, went1, sem_gr1, sem_ge1, sem_sr1,
                  sem_se1, lidx1)

        @pl.when(k % 2 == 0)
        def _():
            compute(pidx0, wrel0, went0, sem_gr0, sem_ge0, sem_sr0, sem_se0,
                    lidx0)

        @pl.when(k % 2 == 1)
        def _():
            compute(pidx1, wrel1, went1, sem_gr1, sem_ge1, sem_sr1, sem_se1,
                    lidx1)

        return carry

    lax.fori_loop(0, nmy, blk, 0)

    @pl.when(nmy >= 1)
    def _():
        @pl.when(nmy % 2 == 1)
        def _():
            pltpu.make_async_copy(wrel0, arel_sh.at[lidx0], sem_sr0).wait()
            pltpu.make_async_copy(went0, aent_sh.at[lidx0], sem_se0).wait()

        @pl.when(nmy % 2 == 0)
        def _():
            pltpu.make_async_copy(wrel1, arel_sh.at[lidx1], sem_sr1).wait()
            pltpu.make_async_copy(went1, aent_sh.at[lidx1], sem_se1).wait()

    @pl.when(nmy >= 2)
    def _():
        @pl.when(nmy % 2 == 0)
        def _():
            pltpu.make_async_copy(wrel0, arel_sh.at[lidx0], sem_sr0).wait()
            pltpu.make_async_copy(went0, aent_sh.at[lidx0], sem_se0).wait()

        @pl.when(nmy % 2 == 1)
        def _():
            pltpu.make_async_copy(wrel1, arel_sh.at[lidx1], sem_sr1).wait()
            pltpu.make_async_copy(went1, aent_sh.at[lidx1], sem_se1).wait()

    plsc.subcore_barrier()

    r0 = pl.multiple_of(sid * (GSC // NS), 8)
    o0 = pl.multiple_of(cg0 + sid * (GSC // NS), 8)
    pltpu.sync_copy(arel_sh.at[pl.ds(r0, GSC // NS)],
                    srel_out.at[pl.ds(o0, GSC // NS)])
    pltpu.sync_copy(aent_sh.at[pl.ds(r0, GSC // NS)],
                    sent_o_out.at[pl.ds(o0, GSC // NS)])


_sc_agg = pl.kernel(
    _sc_body,
    out_type=[
        jax.ShapeDtypeStruct((N_GROUPS, H), jnp.float32),
        jax.ShapeDtypeStruct((N_GROUPS, H), jnp.float32),
        jax.ShapeDtypeStruct((N_GROUPS * 16,), jnp.float32),
        jax.ShapeDtypeStruct((B, H), jnp.float32),
    ],
    mesh=plsc.VectorSubcoreMesh(core_axis_name="c", subcore_axis_name="s"),
    compiler_params=pltpu.CompilerParams(needs_layout_passes=False),
    scratch_types=[
        pltpu.VMEM((4, K), jnp.int32),
        pltpu.VMEM((4, K), jnp.int32),
        pltpu.VMEM((K, H), jnp.float32),
        pltpu.VMEM((K, H), jnp.float32),
        pltpu.VMEM((K, H), jnp.float32),
        pltpu.VMEM((K, H), jnp.float32),
        pltpu.VMEM((K,), jnp.int32),
        pltpu.VMEM((K,), jnp.int32),
        pltpu.VMEM((CNT_N,), jnp.float32),
        pltpu.VMEM((16,), jnp.int32),
        pltpu.VMEM((SROWS,), jnp.int32),
        pltpu.VMEM((SROWS, H), jnp.float32),
        pltpu.VMEM((K,), jnp.int32),
        pltpu.VMEM((GSC // NS - K,), jnp.int32),
        pltpu.VMEM_SHARED((ASH_ROWS, H), jnp.float32),
        pltpu.VMEM_SHARED((ASH_ROWS, H), jnp.float32),
        pltpu.SemaphoreType.DMA,
        pltpu.SemaphoreType.DMA,
        pltpu.SemaphoreType.DMA,
        pltpu.SemaphoreType.DMA,
        pltpu.SemaphoreType.DMA,
        pltpu.SemaphoreType.DMA,
        pltpu.SemaphoreType.DMA,
        pltpu.SemaphoreType.DMA,
        pltpu.SemaphoreType.DMA,
        pltpu.SemaphoreType.DMA,
        pltpu.SemaphoreType.DMA,
    ],
)

def _tc_body(srel_ref, sent_s_ref, cnt_ref, sent_ref, lnr_ref,
             wt_ref, b_ref, out_ref):
    t = pl.program_id(0)
    cnt = jnp.maximum(
        jnp.sum(cnt_ref[...].reshape(B, 16), axis=1, keepdims=True), 1.0)
    mean = jnp.concatenate([srel_ref[...].reshape(B, H),
                            sent_s_ref[...].reshape(B, H)], axis=1) / cnt
    h = jnp.dot(mean, wt_ref[...], preferred_element_type=jnp.float32)
    h = jnp.maximum(h + b_ref[...], 0.0)
    tf = lax.convert_element_type(t, jnp.float32)
    mask = (tf < lnr_ref[...]).astype(jnp.float32)
    out = jnp.concatenate([h, sent_ref[...]], axis=1) * mask
    out_ref[...] = out.reshape(1, B, 3 * H)


_tc_pack = pl.pallas_call(
    _tc_body,
    grid=(SEQ_LEN,),
    in_specs=[
        pl.BlockSpec((1, B, H), lambda t: (t, 0, 0)),
        pl.BlockSpec((1, B, H), lambda t: (t, 0, 0)),
        pl.BlockSpec((1, B, 16), lambda t: (t, 0, 0)),
        pl.BlockSpec((B, H), lambda t: (0, 0)),
        pl.BlockSpec((B, 1), lambda t: (0, 0)),
        pl.BlockSpec((2 * H, 2 * H), lambda t: (0, 0)),
        pl.BlockSpec((1, 2 * H), lambda t: (0, 0)),
    ],
    out_specs=pl.BlockSpec((1, B, 3 * H), lambda t: (t, 0, 0)),
    out_shape=jax.ShapeDtypeStruct((SEQ_LEN, B, 3 * H), jnp.float32),
)


def kernel(obj_idx, rel_idx, rel_weights, segment_ids, len_non_zero, s_idx,
           ent_embeds, rel_embeds, W, b):
    obj_idx = obj_idx.astype(jnp.int32)
    rel_idx = rel_idx.astype(jnp.int32)
    seg = segment_ids.astype(jnp.int32)
    s_idx = s_idx.astype(jnp.int32)
    w = rel_weights.astype(jnp.float32)

    zpad_i = jnp.zeros((K,), jnp.int32)
    pidx = jnp.stack([
        jnp.concatenate([obj_idx, zpad_i]),
        jnp.concatenate([rel_idx, zpad_i]),
        jnp.concatenate([seg, jnp.full((K,), SEG_PAD, jnp.int32)]),
        jnp.concatenate([lax.bitcast_convert_type(w, jnp.int32), zpad_i]),
    ])

    bounds = jnp.searchsorted(seg, jnp.arange(NW + 1, dtype=jnp.int32) * GPW,
                              method="compare_all")
    starts = ((bounds[:NW] // K) * K).astype(jnp.int32)
    ends = bounds[1:].astype(jnp.int32)
    scb = jnp.zeros((16,), jnp.int32)
    scb = scb.at[0].set((bounds[0] // K) * K).at[1].set(bounds[NS])
    scb = scb.at[2].set((bounds[NS] // K) * K).at[3].set(bounds[NW])

    tperm = jnp.zeros((N_GROUPS,), jnp.int32)

    srel, sent_s, cnt16, sent = _sc_agg(
        pidx, starts, ends, scb, s_idx,
        jnp.zeros((ZROWS, H), jnp.float32), tperm, ent_embeds, rel_embeds)
    srel_t = srel.reshape(B, SEQ_LEN, H).transpose(1, 0, 2)
    sent_s_t = sent_s.reshape(B, SEQ_LEN, H).transpose(1, 0, 2)
    cnt_t = (cnt16.reshape(B, SEQ_LEN, 16).transpose(1, 0, 2))
    lnr = jnp.maximum(len_non_zero, 1).astype(jnp.float32)[:, None]
    out = _tc_pack(srel_t, sent_s_t, cnt_t, sent, lnr, W.T, b[None, :])
    return jnp.transpose(out, (1, 0, 2))
